# Initial kernel scaffold; baseline (speedup 1.0000x reference)
import jax
import jax.numpy as jnp
from jax.experimental import pallas as pl


def kernel(x, tok_table, pos_table):
    raise NotImplementedError("write your pallas kernel here")


# speedup vs baseline: 4.3662x; 4.3662x over previous
import functools

import jax
import jax.numpy as jnp
from jax import lax
from jax.experimental import pallas as pl
from jax.experimental.pallas import tpu as pltpu
from jax.experimental.pallas import tpu_sc as plsc

B, T, E = 1024, 200, 128
NC, NS = 2, 16
NW = NC * NS
ROWS_PW = B // NW
N = B * T
IA, IB = 128, T - 128

_mesh = plsc.VectorSubcoreMesh(core_axis_name="c", subcore_axis_name="s")


@functools.partial(
    pl.kernel,
    out_type=jax.ShapeDtypeStruct((N, E), jnp.float32),
    mesh=_mesh,
    scratch_types=[
        pltpu.VMEM_SHARED((T, E), jnp.float32),
        pltpu.VMEM((T, E), jnp.float32),
        pltpu.VMEM((IA,), jnp.int32),
        pltpu.VMEM((IB,), jnp.int32),
        pltpu.SemaphoreType.DMA,
        pltpu.SemaphoreType.DMA,
    ],
)
def _emb_kernel(x_hbm, tok_hbm, pos_hbm, out_hbm,
                pos_v, rows_v, idxa_v, idxb_v, sema, semb):
    sid = lax.axis_index("s")
    wid = sid * NC + lax.axis_index("c")

    @pl.when(sid == 0)
    def _stage_pos():
        pltpu.sync_copy(pos_hbm.at[pl.ds(0, T)], pos_v)

    plsc.subcore_barrier()

    def body(i, _):
        base = (wid * ROWS_PW + i) * T
        pltpu.sync_copy(x_hbm.at[pl.ds(base, IA)], idxa_v)
        pltpu.sync_copy(x_hbm.at[pl.ds(base + IA, IB)], idxb_v)
        pltpu.sync_copy(pos_v, rows_v)
        ca = pltpu.async_copy(tok_hbm.at[idxa_v], rows_v.at[pl.ds(0, IA)],
                              sema, add=True)
        cb = pltpu.async_copy(tok_hbm.at[idxb_v], rows_v.at[pl.ds(IA, IB)],
                              semb, add=True)
        ca.wait()
        cb.wait()
        pltpu.sync_copy(rows_v, out_hbm.at[pl.ds(base, T)])
        return 0

    lax.fori_loop(0, ROWS_PW, body, 0)


def kernel(x, tok_table, pos_table):
    out = _emb_kernel(x.reshape(-1).astype(jnp.int32), tok_table, pos_table)
    return out.reshape(B, T, E)


# speedup vs baseline: 5.1876x; 1.1881x over previous
import functools

import jax
import jax.numpy as jnp
from jax import lax
from jax.experimental import pallas as pl
from jax.experimental.pallas import tpu as pltpu
from jax.experimental.pallas import tpu_sc as plsc

B, T, E = 1024, 200, 128
NC, NS = 2, 16
NW = NC * NS
ROWS_PW = B // NW
N = B * T
IA, IB = 128, T - 128

_mesh = plsc.VectorSubcoreMesh(core_axis_name="c", subcore_axis_name="s")


@functools.partial(
    pl.kernel,
    out_type=jax.ShapeDtypeStruct((N, E), jnp.float32),
    mesh=_mesh,
    scratch_types=[
        pltpu.VMEM_SHARED((T, E), jnp.float32),
        pltpu.VMEM((T, E), jnp.float32),
        pltpu.VMEM((T, E), jnp.float32),
        pltpu.VMEM((IA,), jnp.int32),
        pltpu.VMEM((IA,), jnp.int32),
        pltpu.VMEM((IB,), jnp.int32),
        pltpu.VMEM((IB,), jnp.int32),
        pltpu.SemaphoreType.DMA,
        pltpu.SemaphoreType.DMA,
        pltpu.SemaphoreType.DMA,
        pltpu.SemaphoreType.DMA,
    ],
)
def _emb_kernel(x_hbm, tok_hbm, pos_hbm, out_hbm,
                pos_sh, rows0, rows1, idxa0, idxa1, idxb0, idxb1,
                sem_g0, sem_g1, sem_w0, sem_w1):
    rows = (rows0, rows1)
    idxa = (idxa0, idxa1)
    idxb = (idxb0, idxb1)
    sem_g = (sem_g0, sem_g1)
    sem_w = (sem_w0, sem_w1)

    sid = lax.axis_index("s")
    wid = sid * NC + lax.axis_index("c")

    @pl.when(sid == 0)
    def _stage_pos():
        pltpu.sync_copy(pos_hbm.at[pl.ds(0, T)], pos_sh)

    plsc.subcore_barrier()

    cbase = wid * ROWS_PW

    def stage_and_gather(p, base):
        pltpu.sync_copy(x_hbm.at[pl.ds(base, IA)], idxa[p])
        pltpu.sync_copy(x_hbm.at[pl.ds(base + IA, IB)], idxb[p])
        pltpu.sync_copy(pos_sh, rows[p])
        return (
            pltpu.async_copy(tok_hbm.at[idxa[p]], rows[p].at[pl.ds(0, IA)],
                             sem_g[p], add=True),
            pltpu.async_copy(tok_hbm.at[idxb[p]], rows[p].at[pl.ds(IA, IB)],
                             sem_g[p], add=True),
        )

    def body(g, _):
        base = (cbase + 2 * g) * T
        gs0 = stage_and_gather(0, base)
        gs1 = stage_and_gather(1, base + T)
        for c in gs0:
            c.wait()
        w0 = pltpu.async_copy(rows[0], out_hbm.at[pl.ds(base, T)], sem_w[0])
        for c in gs1:
            c.wait()
        w1 = pltpu.async_copy(rows[1], out_hbm.at[pl.ds(base + T, T)], sem_w[1])
        w0.wait()
        w1.wait()
        return 0

    lax.fori_loop(0, ROWS_PW // 2, body, 0)


def kernel(x, tok_table, pos_table):
    out = _emb_kernel(x.reshape(-1).astype(jnp.int32), tok_table, pos_table)
    return out.reshape(B, T, E)


# speedup vs baseline: 6.7404x; 1.2993x over previous
import functools

import jax
import jax.numpy as jnp
from jax import lax
from jax.experimental import pallas as pl
from jax.experimental.pallas import tpu as pltpu
from jax.experimental.pallas import tpu_sc as plsc

B, T, E = 1024, 200, 128
NC, NS = 2, 16
NW = NC * NS
ROWS_PW = B // NW
N = B * T
IA, IB = 128, T - 128

_mesh = plsc.VectorSubcoreMesh(core_axis_name="c", subcore_axis_name="s")


@functools.partial(
    pl.kernel,
    out_type=jax.ShapeDtypeStruct((N, E), jnp.float32),
    mesh=_mesh,
    scratch_types=[
        pltpu.VMEM_SHARED((T, E), jnp.float32),
        pltpu.VMEM((T, E), jnp.float32),
        pltpu.VMEM((T, E), jnp.float32),
        pltpu.VMEM((ROWS_PW * T,), jnp.int32),
        pltpu.SemaphoreType.DMA,
        pltpu.SemaphoreType.DMA,
        pltpu.SemaphoreType.DMA,
        pltpu.SemaphoreType.DMA,
    ],
)
def _emb_kernel(x_hbm, tok_hbm, pos_hbm, out_hbm,
                pos_sh, rows0, rows1, idx_v,
                sem_g0, sem_g1, sem_w0, sem_w1):
    rows = (rows0, rows1)
    sem_g = (sem_g0, sem_g1)
    sem_w = (sem_w0, sem_w1)

    sid = lax.axis_index("s")
    wid = sid * NC + lax.axis_index("c")

    @pl.when(sid == 0)
    def _stage_pos():
        pltpu.sync_copy(pos_hbm.at[pl.ds(0, T)], pos_sh)

    plsc.subcore_barrier()

    cbase = wid * ROWS_PW

    pltpu.sync_copy(x_hbm.at[pl.ds(cbase * T, ROWS_PW * T)], idx_v)

    def stage_and_gather(p, off):
        pltpu.sync_copy(pos_sh, rows[p])
        return (
            pltpu.async_copy(tok_hbm.at[idx_v.at[pl.ds(off, IA)]],
                             rows[p].at[pl.ds(0, IA)], sem_g[p], add=True),
            pltpu.async_copy(tok_hbm.at[idx_v.at[pl.ds(off + IA, IB)]],
                             rows[p].at[pl.ds(IA, IB)], sem_g[p], add=True),
        )

    def body(g, _):
        base = (cbase + 2 * g) * T
        off = 2 * g * T
        gs0 = stage_and_gather(0, off)
        gs1 = stage_and_gather(1, off + T)
        for c in gs0:
            c.wait()
        w0 = pltpu.async_copy(rows[0], out_hbm.at[pl.ds(base, T)], sem_w[0])
        for c in gs1:
            c.wait()
        w1 = pltpu.async_copy(rows[1], out_hbm.at[pl.ds(base + T, T)], sem_w[1])
        w0.wait()
        w1.wait()
        return 0

    lax.fori_loop(0, ROWS_PW // 2, body, 0)


def kernel(x, tok_table, pos_table):
    out = _emb_kernel(x.reshape(-1).astype(jnp.int32), tok_table, pos_table)
    return out.reshape(B, T, E)


# speedup vs baseline: 7.4794x; 1.1096x over previous
import functools

import jax
import jax.numpy as jnp
from jax import lax
from jax.experimental import pallas as pl
from jax.experimental.pallas import tpu as pltpu
from jax.experimental.pallas import tpu_sc as plsc

B, T, E = 1024, 200, 128
NC, NS = 2, 16
NW = NC * NS
ROWS_PW = B // NW
N = B * T
IA, IB = 128, T - 128

_mesh = plsc.VectorSubcoreMesh(core_axis_name="c", subcore_axis_name="s")


@functools.partial(
    pl.kernel,
    out_type=jax.ShapeDtypeStruct((N, E), jnp.float32),
    mesh=_mesh,
    scratch_types=[
        pltpu.VMEM_SHARED((T, E), jnp.float32),
        pltpu.VMEM((T, E), jnp.float32),
        pltpu.VMEM((T, E), jnp.float32),
        pltpu.VMEM((ROWS_PW * T,), jnp.int32),
        pltpu.SemaphoreType.DMA,
        pltpu.SemaphoreType.DMA,
        pltpu.SemaphoreType.DMA,
        pltpu.SemaphoreType.DMA,
    ],
)
def _emb_kernel(x_hbm, tok_hbm, pos_hbm, out_hbm,
                pos_sh, rows0, rows1, idx_v,
                sem_g0, sem_g1, sem_w0, sem_w1):
    rows = (rows0, rows1)
    sem_g = (sem_g0, sem_g1)
    sem_w = (sem_w0, sem_w1)

    sid = lax.axis_index("s")
    wid = sid * NC + lax.axis_index("c")

    @pl.when(sid == 0)
    def _stage_pos():
        pltpu.sync_copy(pos_hbm.at[pl.ds(0, T)], pos_sh)

    plsc.subcore_barrier()

    cbase = wid * ROWS_PW

    pltpu.sync_copy(x_hbm.at[pl.ds(cbase * T, ROWS_PW * T)], idx_v)

    def stage_and_gather(p, off):
        pltpu.sync_copy(pos_sh, rows[p])
        return (
            pltpu.async_copy(tok_hbm.at[idx_v.at[pl.ds(off, IA)]],
                             rows[p].at[pl.ds(0, IA)], sem_g[p], add=True),
            pltpu.async_copy(tok_hbm.at[idx_v.at[pl.ds(off + IA, IB)]],
                             rows[p].at[pl.ds(IA, IB)], sem_g[p], add=True),
        )

    def start_writes(base, gs0, gs1):
        for c in gs0:
            c.wait()
        pltpu.async_copy(rows[0], out_hbm.at[pl.ds(base, T)], sem_w[0])
        for c in gs1:
            c.wait()
        pltpu.async_copy(rows[1], out_hbm.at[pl.ds(base + T, T)], sem_w[1])

    start_writes(cbase * T, stage_and_gather(0, 0), stage_and_gather(1, T))

    def body(g, _):
        base = (cbase + 2 * g) * T
        off = 2 * g * T
        pltpu.make_async_copy(rows[0], out_hbm.at[pl.ds(base - 2 * T, T)],
                              sem_w[0]).wait()
        gs0 = stage_and_gather(0, off)
        pltpu.make_async_copy(rows[1], out_hbm.at[pl.ds(base - T, T)],
                              sem_w[1]).wait()
        gs1 = stage_and_gather(1, off + T)
        start_writes(base, gs0, gs1)
        return 0

    lax.fori_loop(1, ROWS_PW // 2, body, 0)

    last = (cbase + ROWS_PW - 2) * T
    pltpu.make_async_copy(rows[0], out_hbm.at[pl.ds(last, T)], sem_w[0]).wait()
    pltpu.make_async_copy(rows[1], out_hbm.at[pl.ds(last + T, T)], sem_w[1]).wait()


def kernel(x, tok_table, pos_table):
    out = _emb_kernel(x.reshape(-1).astype(jnp.int32), tok_table, pos_table)
    return out.reshape(B, T, E)
